# Initial kernel scaffold; baseline (speedup 1.0000x reference)
import jax
import jax.numpy as jnp
from jax.experimental import pallas as pl


def kernel(src_ids, tgt_ids, src_table, tgt_table):
    raise NotImplementedError("write your pallas kernel here")


# speedup vs baseline: 3.9386x; 3.9386x over previous
import functools

import jax
import jax.numpy as jnp
from jax import lax
from jax.experimental import pallas as pl
from jax.experimental.pallas import tpu as pltpu
from jax.experimental.pallas import tpu_sc as plsc


@functools.cache
def _build(n, d, chunk, n_chunks, num_cores):
    mesh = plsc.VectorSubcoreMesh(core_axis_name="c", subcore_axis_name="s")

    @functools.partial(
        pl.kernel,
        mesh=mesh,
        out_type=(
            jax.ShapeDtypeStruct((n, d), jnp.float32),
            jax.ShapeDtypeStruct((n, d), jnp.float32),
        ),
        scratch_types=[
            pltpu.VMEM((chunk,), jnp.int32),
            pltpu.VMEM((chunk, d), jnp.float32),
            pltpu.SemaphoreType.DMA,
        ],
        compiler_params=pltpu.CompilerParams(use_tc_tiling_on_sc=False),
    )
    def k(src_idx, tgt_idx, src_tab, tgt_tab, src_out, tgt_out,
          idx_v, rows_v, sem):
        wid = lax.axis_index("s") * num_cores + lax.axis_index("c")
        base = wid * (n_chunks * chunk)

        def body(i, carry):
            off = base + i * chunk
            pltpu.sync_copy(src_idx.at[pl.ds(off, chunk)], idx_v)
            pltpu.async_copy(src_tab.at[idx_v], rows_v, sem).wait()
            pltpu.sync_copy(rows_v, src_out.at[pl.ds(off, chunk)])
            pltpu.sync_copy(tgt_idx.at[pl.ds(off, chunk)], idx_v)
            pltpu.async_copy(tgt_tab.at[idx_v], rows_v, sem).wait()
            pltpu.sync_copy(rows_v, tgt_out.at[pl.ds(off, chunk)])
            return carry

        lax.fori_loop(0, n_chunks, body, 0)

    return k


def kernel(src_ids, tgt_ids, src_table, tgt_table):
    b, l = src_ids.shape
    d = src_table.shape[1]
    n = b * l
    info = plsc.get_sparse_core_info()
    nw = info.num_cores * info.num_subcores
    chunk = 128
    assert n % (nw * chunk) == 0
    n_chunks = n // (nw * chunk)
    k = _build(n, d, chunk, n_chunks, info.num_cores)
    src_flat = src_ids.reshape(n).astype(jnp.int32)
    tgt_flat = tgt_ids.reshape(n).astype(jnp.int32)
    src_out, tgt_out = k(src_flat, tgt_flat, src_table, tgt_table)
    return src_out.reshape(b, l, d), tgt_out.reshape(b, l, d)


# speedup vs baseline: 4.9693x; 1.2617x over previous
import functools

import jax
import jax.numpy as jnp
from jax import lax
from jax.experimental import pallas as pl
from jax.experimental.pallas import tpu as pltpu
from jax.experimental.pallas import tpu_sc as plsc

_CHUNK = 128
_K = 5


@functools.cache
def _build(n, d, n_chunks, num_cores, num_subcores):
    chunk, k_grp = _CHUNK, _K
    n_groups = n_chunks // k_grp
    assert n_chunks == n_groups * k_grp and n_groups >= 2
    n_per_w = n_chunks * chunk
    half = k_grp * chunk
    mesh = plsc.VectorSubcoreMesh(core_axis_name="c", subcore_axis_name="s")

    @functools.partial(
        pl.kernel,
        mesh=mesh,
        out_type=(
            jax.ShapeDtypeStruct((n, d), jnp.float32),
            jax.ShapeDtypeStruct((n, d), jnp.float32),
        ),
        scratch_types=[
            pltpu.VMEM((n_chunks, chunk), jnp.int32),
            pltpu.VMEM((n_chunks, chunk), jnp.int32),
            pltpu.VMEM((2 * k_grp * chunk, d), jnp.float32),
            pltpu.SemaphoreType.DMA,
            pltpu.SemaphoreType.DMA,
            pltpu.SemaphoreType.DMA,
            pltpu.SemaphoreType.DMA,
        ],
        compiler_params=pltpu.CompilerParams(use_tc_tiling_on_sc=False),
    )
    def k(src_idx, tgt_idx, src_tab, tgt_tab, src_out, tgt_out,
          sidx_v, tidx_v, rows_v, sg_src, sw_src, sg_tgt, sw_tgt):
        wid = lax.axis_index("s") * num_cores + lax.axis_index("c")
        base = wid * n_per_w
        crow = wid * n_chunks

        pltpu.sync_copy(src_idx.at[pl.ds(crow, n_chunks)], sidx_v)
        pltpu.sync_copy(tgt_idx.at[pl.ds(crow, n_chunks)], tidx_v)

        def do_table(tab, out, idx_v, sem_g, sem_w):
            def issue_gathers(g, buf_base):
                for b in range(k_grp):
                    pltpu.async_copy(
                        tab.at[idx_v.at[g * k_grp + b]],
                        rows_v.at[pl.ds(buf_base + b * chunk, chunk)],
                        sem_g)

            def wait_gathers():
                for _ in range(k_grp):
                    pltpu.make_async_copy(
                        tab.at[pl.ds(0, chunk)],
                        rows_v.at[pl.ds(0, chunk)], sem_g).wait()

            def issue_writes(g, buf_base):
                for b in range(k_grp):
                    pltpu.async_copy(
                        rows_v.at[pl.ds(buf_base + b * chunk, chunk)],
                        out.at[pl.ds(base + (g * k_grp + b) * chunk, chunk)],
                        sem_w)

            def wait_writes():
                for _ in range(k_grp):
                    pltpu.make_async_copy(
                        rows_v.at[pl.ds(0, chunk)],
                        out.at[pl.ds(base, chunk)], sem_w).wait()

            issue_gathers(0, 0)
            wait_gathers()
            issue_writes(0, 0)
            issue_gathers(1, half)

            def body(t, carry):
                buf = lax.rem(t, 2) * half
                wait_gathers()
                wait_writes()
                issue_writes(t, buf)
                issue_gathers(t + 1, half - buf)
                return carry

            lax.fori_loop(1, n_groups - 1, body, 0)

            last = n_groups - 1
            wait_gathers()
            wait_writes()
            issue_writes(last, lax.rem(last, 2) * half)
            wait_writes()

        do_table(src_tab, src_out, sidx_v, sg_src, sw_src)
        do_table(tgt_tab, tgt_out, tidx_v, sg_tgt, sw_tgt)

    return k


def kernel(src_ids, tgt_ids, src_table, tgt_table):
    b, l = src_ids.shape
    d = src_table.shape[1]
    n = b * l
    info = plsc.get_sparse_core_info()
    nw = info.num_cores * info.num_subcores
    assert n % (nw * _CHUNK) == 0
    n_chunks = n // (nw * _CHUNK)
    k = _build(n, d, n_chunks, info.num_cores, info.num_subcores)
    src_flat = src_ids.reshape(n // _CHUNK, _CHUNK).astype(jnp.int32)
    tgt_flat = tgt_ids.reshape(n // _CHUNK, _CHUNK).astype(jnp.int32)
    src_out, tgt_out = k(src_flat, tgt_flat, src_table, tgt_table)
    return src_out.reshape(b, l, d), tgt_out.reshape(b, l, d)
